# baseline (speedup 1.0000x reference)
import jax
import jax.numpy as jnp
import numpy as np
from jax.experimental import pallas as pl
from jax.experimental.pallas import tpu as pltpu

D = 768
NUM_HEADS = 12
HEAD_DIM = D // NUM_HEADS
ATT_SCALE = HEAD_DIM ** (-0.5)
FF = 3072
FF_TILE = 1536
N_LAYERS = 12
LN_EPS = 1e-5
BB = 16
NCHUNK = 2
S = 80


def _norm(x):
    mu = jnp.mean(x, axis=-1, keepdims=True)
    var = jnp.mean(jnp.square(x - mu), axis=-1, keepdims=True)
    return (x - mu) * jax.lax.rsqrt(var + LN_EPS)


def _gelu_tanh(x):
    c = 0.7978845608028654
    c2 = c * 0.044715
    v = x * (c + c2 * (x * x))
    h = 0.5 * x
    return h + h * jnp.tanh(v)


def _attn_pass(xs, mask2d, w_in, b_in, wo_p, bo):
    nc = len(xs)
    abs_ = []
    for x in xs:
        rows = x.shape[0]
        cb = rows // S
        xn = _norm(x).astype(jnp.bfloat16)
        ab = (jnp.dot(xn, w_in, preferred_element_type=jnp.float32) + b_in)
        abs_.append(ab.astype(jnp.bfloat16).reshape(cb, S, 2 * NUM_HEADS * 128))

    ctx_heads = [[] for _ in range(nc)]
    for h in range(NUM_HEADS):
        for c in range(nc):
            ab = abs_[c]
            a_h = ab[:, :, 128 * h:128 * h + 128]
            b_h = ab[:, :, 1536 + 128 * h:1536 + 128 * h + 128]
            sc = jax.lax.dot_general(a_h, b_h, (((2,), (2,)), ((0,), (0,))),
                                     preferred_element_type=jnp.float32)
            e = jnp.exp(sc + mask2d[None])
            p = (e / jnp.sum(e, axis=-1, keepdims=True)).astype(jnp.bfloat16)
            ctx_heads[c].append(jax.lax.dot_general(
                p, b_h, (((2,), (1,)), ((0,), (0,))),
                preferred_element_type=jnp.float32).astype(jnp.bfloat16))

    outs = []
    for c in range(nc):
        rows = xs[c].shape[0]
        ctx = jnp.concatenate(ctx_heads[c], axis=-1)
        ctx = ctx.reshape(rows, NUM_HEADS * 128)
        attn = jnp.dot(ctx, wo_p, preferred_element_type=jnp.float32) + bo
        outs.append(xs[c] + attn)
    return outs


def _mlp_chunk(x, w1, b1, w2, b2):
    xn2 = _norm(x).astype(jnp.bfloat16)
    acc = x + b2
    for t in range(FF // FF_TILE):
        fo = t * FF_TILE
        ht = jnp.dot(xn2, w1[:, fo:fo + FF_TILE],
                     preferred_element_type=jnp.float32) + b1[:, fo:fo + FF_TILE]
        ht = _gelu_tanh(ht).astype(jnp.bfloat16)
        acc = acc + jnp.dot(ht, w2[fo:fo + FF_TILE, :],
                            preferred_element_type=jnp.float32)
    return acc


def _encoder_kernel(x_hbm, mask_ref, w_in_ref, b_in_ref, wo_ref, bo_ref,
                    w1_ref, b1_ref, w2_ref, b2_ref, out_ref, dma_sem):
    layer = pl.program_id(1)

    @pl.when(layer == 0)
    def _():
        nb = pl.program_id(0)
        cp = pltpu.make_async_copy(x_hbm.at[pl.ds(nb * BB, BB)], out_ref,
                                   dma_sem)
        cp.start()
        cp.wait()

    mask2d = mask_ref[0, 0]
    cb = BB // NCHUNK
    xs = [out_ref[c * cb:(c + 1) * cb].reshape(cb * S, D)
          for c in range(NCHUNK)]
    xs = _attn_pass(xs, mask2d, w_in_ref[0], b_in_ref[0],
                    wo_ref[0], bo_ref[0])
    for c in range(NCHUNK):
        y = _mlp_chunk(xs[c], w1_ref[0], b1_ref[0], w2_ref[0], b2_ref[0])
        out_ref[c * cb:(c + 1) * cb] = y.reshape(cb, S, D)


def kernel(hidden, mask, ln1_g, ln1_b, qkv_w, qkv_b, wo, bo,
           ln2_g, ln2_b, w1, b1, w2, b2):
    B, S_, _ = hidden.shape
    nb = B // BB
    L = N_LAYERS

    hh = np.arange(NUM_HEADS)[:, None]
    rr = np.arange(HEAD_DIM)[None, :]
    perm = np.zeros((2 * NUM_HEADS, 128), np.int32)
    cmask = np.zeros((2 * NUM_HEADS, 128), np.float32)
    perm[:NUM_HEADS, :HEAD_DIM] = 64 * hh + rr
    cmask[:NUM_HEADS, :HEAD_DIM] = ATT_SCALE
    perm[NUM_HEADS:, :HEAD_DIM] = D + 64 * hh + rr
    cmask[NUM_HEADS:, :HEAD_DIM] = 1.0
    perm[NUM_HEADS:, HEAD_DIM:] = 2 * D + 64 * hh + rr
    cmask[NUM_HEADS:, HEAD_DIM:] = 1.0
    perm = perm.reshape(-1)
    cmask = cmask.reshape(-1)

    w_in = (qkv_w[:, :, perm] * ln1_g[:, 0, :, None]
            * cmask).astype(jnp.bfloat16)
    bvec1 = ln1_b[:, 0, :] / ln1_g[:, 0, :]
    b_in = (qkv_b[:, 0, perm] * cmask
            + jnp.einsum('ld,ldo->lo', bvec1, w_in,
                         preferred_element_type=jnp.float32))
    b_in = b_in.reshape(L, 1, 2 * NUM_HEADS * 128)

    rowperm = (64 * (np.arange(NUM_HEADS * 128) // 128)
               + np.maximum(np.arange(NUM_HEADS * 128) % 128 - 64, 0))
    rmask = ((np.arange(NUM_HEADS * 128) % 128) >= 64).astype(np.float32)
    wo_p = (wo[:, rowperm, :] * rmask[None, :, None]).astype(jnp.bfloat16)

    w1_f = (w1 * ln2_g[:, 0, :, None]).astype(jnp.bfloat16)
    bvec2 = ln2_b[:, 0, :] / ln2_g[:, 0, :]
    b1_f = (b1[:, 0, :] + jnp.einsum('ld,ldo->lo', bvec2, w1_f,
                                     preferred_element_type=jnp.float32))
    b1_f = b1_f.reshape(L, 1, FF)
    w2 = w2.astype(jnp.bfloat16)

    return pl.pallas_call(
        _encoder_kernel,
        out_shape=jax.ShapeDtypeStruct((B, S_, D), jnp.float32),
        grid_spec=pltpu.PrefetchScalarGridSpec(
            num_scalar_prefetch=0,
            grid=(nb, N_LAYERS),
            in_specs=[
                pl.BlockSpec(memory_space=pl.ANY),
                pl.BlockSpec((1, 1, S_, S_), lambda b, l: (0, 0, 0, 0)),
                pl.BlockSpec((1, D, 2 * NUM_HEADS * 128),
                             lambda b, l: (l, 0, 0)),
                pl.BlockSpec((1, 1, 2 * NUM_HEADS * 128),
                             lambda b, l: (l, 0, 0)),
                pl.BlockSpec((1, NUM_HEADS * 128, D),
                             lambda b, l: (l, 0, 0)),
                pl.BlockSpec((1, 1, D), lambda b, l: (l, 0, 0)),
                pl.BlockSpec((1, D, FF), lambda b, l: (l, 0, 0)),
                pl.BlockSpec((1, 1, FF), lambda b, l: (l, 0, 0)),
                pl.BlockSpec((1, FF, D), lambda b, l: (l, 0, 0)),
                pl.BlockSpec((1, 1, D), lambda b, l: (l, 0, 0)),
            ],
            out_specs=pl.BlockSpec((BB, S_, D), lambda b, l: (b, 0, 0)),
            scratch_shapes=[pltpu.SemaphoreType.DMA],
        ),
        compiler_params=pltpu.CompilerParams(
            dimension_semantics=("parallel", "arbitrary"),
            vmem_limit_bytes=56 * 1024 * 1024,
        ),
    )(hidden, mask, w_in, b_in, wo_p, bo, w1_f, b1_f, w2, b2)


# speedup vs baseline: 1.2974x; 1.2974x over previous
import jax
import jax.numpy as jnp
from jax.experimental import pallas as pl
from jax.experimental.pallas import tpu as pltpu

D = 768
NUM_HEADS = 12
HEAD_DIM = D // NUM_HEADS
ATT_SCALE = HEAD_DIM ** (-0.5)
FF = 3072
FF_TILE = 1536
N_LAYERS = 12
LN_EPS = 1e-5
BB = 16
NCHUNK = 2
S = 80


def _norm(x):
    mu = jnp.mean(x, axis=-1, keepdims=True)
    var = jnp.mean(jnp.square(x - mu), axis=-1, keepdims=True)
    return (x - mu) * jax.lax.rsqrt(var + LN_EPS)


def _gelu_tanh(x):
    c = 0.7978845608028654
    c2 = c * 0.044715
    v = x * (c + c2 * (x * x))
    h = 0.5 * x
    return h + h * jnp.tanh(v)


def _attn_pass(xs, mask2d, w_qkv, b_qkv, bv1, wo, bo):
    nc = len(xs)
    qkvs = []
    for x in xs:
        rows = x.shape[0]
        cb = rows // S
        xn = (_norm(x) + bv1).astype(jnp.bfloat16)
        qkv = jnp.dot(xn, w_qkv, preferred_element_type=jnp.float32) + b_qkv
        qkvs.append(qkv.reshape(cb, S, 3 * D))

    ctx_heads = [[] for _ in range(nc)]
    for h in range(NUM_HEADS):
        lo = h * HEAD_DIM
        for c in range(nc):
            qkv = qkvs[c]
            qh = qkv[:, :, lo:lo + HEAD_DIM].astype(jnp.bfloat16)
            kh = qkv[:, :, D + lo:D + lo + HEAD_DIM].astype(jnp.bfloat16)
            vh = qkv[:, :, 2 * D + lo:2 * D + lo + HEAD_DIM].astype(jnp.bfloat16)
            sc = jax.lax.dot_general(qh, kh, (((2,), (2,)), ((0,), (0,))),
                                     preferred_element_type=jnp.float32)
            e = jnp.exp(sc + mask2d[None])
            p = (e / jnp.sum(e, axis=-1, keepdims=True)).astype(jnp.bfloat16)
            ctx_heads[c].append(jax.lax.dot_general(
                p, vh, (((2,), (1,)), ((0,), (0,))),
                preferred_element_type=jnp.float32).astype(jnp.bfloat16))

    outs = []
    for c in range(nc):
        rows = xs[c].shape[0]
        ctx = jnp.concatenate(ctx_heads[c], axis=-1)
        ctx = ctx.reshape(rows, D)
        attn = jnp.dot(ctx, wo, preferred_element_type=jnp.float32) + bo
        outs.append(xs[c] + attn)
    return outs


def _mlp_chunk(x, bv2, w1, b1, w2, b2):
    xn2 = (_norm(x) + bv2).astype(jnp.bfloat16)
    acc = x + b2
    for t in range(FF // FF_TILE):
        fo = t * FF_TILE
        ht = jnp.dot(xn2, w1[:, fo:fo + FF_TILE],
                     preferred_element_type=jnp.float32) + b1[:, fo:fo + FF_TILE]
        ht = _gelu_tanh(ht).astype(jnp.bfloat16)
        acc = acc + jnp.dot(ht, w2[fo:fo + FF_TILE, :],
                            preferred_element_type=jnp.float32)
    return acc


def _encoder_kernel(x_hbm, mask_ref, w_qkv_ref, b_qkv_ref, bv1_ref, wo_ref,
                    bo_ref, bv2_ref, w1_ref, b1_ref, w2_ref, b2_ref,
                    out_ref, dma_sem):
    layer = pl.program_id(1)

    @pl.when(layer == 0)
    def _():
        nb = pl.program_id(0)
        cp = pltpu.make_async_copy(x_hbm.at[pl.ds(nb * BB, BB)], out_ref,
                                   dma_sem)
        cp.start()
        cp.wait()

    mask2d = mask_ref[0, 0]
    cb = BB // NCHUNK
    xs = [out_ref[c * cb:(c + 1) * cb].reshape(cb * S, D)
          for c in range(NCHUNK)]
    xs = _attn_pass(xs, mask2d, w_qkv_ref[0], b_qkv_ref[0], bv1_ref[0],
                    wo_ref[0], bo_ref[0])
    for c in range(NCHUNK):
        y = _mlp_chunk(xs[c], bv2_ref[0], w1_ref[0], b1_ref[0], w2_ref[0],
                       b2_ref[0])
        out_ref[c * cb:(c + 1) * cb] = y.reshape(cb, S, D)


def kernel(hidden, mask, ln1_g, ln1_b, qkv_w, qkv_b, wo, bo,
           ln2_g, ln2_b, w1, b1, w2, b2):
    B, S_, _ = hidden.shape
    nb = B // BB
    L = N_LAYERS

    cscale = jnp.concatenate(
        [jnp.full((D,), ATT_SCALE, jnp.float32),
         jnp.ones((2 * D,), jnp.float32)])
    w_qkv = (qkv_w * ln1_g[:, 0, :, None] * cscale).astype(jnp.bfloat16)
    b_qkv = qkv_b * cscale
    bv1 = ln1_b / ln1_g
    w1_f = (w1 * ln2_g[:, 0, :, None]).astype(jnp.bfloat16)
    bv2 = ln2_b / ln2_g
    wo_b = wo.astype(jnp.bfloat16)
    w2_b = w2.astype(jnp.bfloat16)

    return pl.pallas_call(
        _encoder_kernel,
        out_shape=jax.ShapeDtypeStruct((B, S_, D), jnp.float32),
        grid_spec=pltpu.PrefetchScalarGridSpec(
            num_scalar_prefetch=0,
            grid=(nb, N_LAYERS),
            in_specs=[
                pl.BlockSpec(memory_space=pl.ANY),
                pl.BlockSpec((1, 1, S_, S_), lambda b, l: (0, 0, 0, 0)),
                pl.BlockSpec((1, D, 3 * D), lambda b, l: (l, 0, 0)),
                pl.BlockSpec((1, 1, 3 * D), lambda b, l: (l, 0, 0)),
                pl.BlockSpec((1, 1, D), lambda b, l: (l, 0, 0)),
                pl.BlockSpec((1, D, D), lambda b, l: (l, 0, 0)),
                pl.BlockSpec((1, 1, D), lambda b, l: (l, 0, 0)),
                pl.BlockSpec((1, 1, D), lambda b, l: (l, 0, 0)),
                pl.BlockSpec((1, D, FF), lambda b, l: (l, 0, 0)),
                pl.BlockSpec((1, 1, FF), lambda b, l: (l, 0, 0)),
                pl.BlockSpec((1, FF, D), lambda b, l: (l, 0, 0)),
                pl.BlockSpec((1, 1, D), lambda b, l: (l, 0, 0)),
            ],
            out_specs=pl.BlockSpec((BB, S_, D), lambda b, l: (b, 0, 0)),
            scratch_shapes=[pltpu.SemaphoreType.DMA],
        ),
        compiler_params=pltpu.CompilerParams(
            dimension_semantics=("parallel", "arbitrary"),
            vmem_limit_bytes=56 * 1024 * 1024,
        ),
    )(hidden, mask, w_qkv, b_qkv, bv1, wo_b, bo, bv2, w1_f, b1, w2_b, b2)
